# baseline (speedup 1.0000x reference)
import jax
import jax.numpy as jnp
from jax import lax
from jax.experimental import pallas as pl
from jax.experimental.pallas import tpu as pltpu
from jax.experimental.pallas import tpu_sc as plsc

NUM_CORES = 2
NUM_SUBCORES = 16
LANES = 16
NW = NUM_CORES * NUM_SUBCORES

BATCH = 16384
DIM = 32
BPW = BATCH // NW
RING = 16
ROUNDS = BPW // RING
CHUNKS = BPW // LANES


def _bpr_body(uid_hbm, pid_hbm, nid_hbm, uembT_hbm, iembT_hbm,
              outp_hbm, outn_hbm,
              ids_sm, ids_v, ring_v, ucols_v, pcols_v, ncols_v,
              outp_v, outn_v, sems):
    wid = lax.axis_index("s") * NUM_CORES + lax.axis_index("c")
    base = pl.multiple_of(wid * BPW, 128)
    lanes = lax.iota(jnp.int32, LANES)

    def gather_role(id_hbm, tab_hbm, cols_v):
        pltpu.sync_copy(id_hbm.at[pl.ds(base, BPW)], ids_v)

        def load_ids(k, carry):
            vec = ids_v[pl.ds(k * LANES, LANES)]
            for l in range(LANES):
                b = lax.reduce_max(jnp.where(lanes == l, vec, 0), (0,))
                ids_sm[k * LANES + l] = b
            return carry

        lax.fori_loop(0, BPW // LANES, load_ids, 0)

        def enqueue(i, r):
            blk = pl.multiple_of(ids_sm[i] & ~(128 - 1), 128)
            pltpu.async_copy(tab_hbm.at[:, pl.ds(blk, 128)],
                             ring_v.at[r], sems.at[r])

        def extract(i, r):
            pltpu.make_async_copy(tab_hbm.at[:, pl.ds(0, 128)],
                                  ring_v.at[r], sems.at[r]).wait()
            j = jnp.full((LANES,), ids_sm[i] & (128 - 1), jnp.int32)
            col = jnp.full((LANES,), i, jnp.int32)
            v0 = plsc.load_gather(ring_v.at[r], [lanes, j])
            v1 = plsc.load_gather(ring_v.at[r], [lanes + LANES, j])
            plsc.store_scatter(cols_v, [lanes, col], v0)
            plsc.store_scatter(cols_v, [lanes + LANES, col], v1)

        for r in range(RING):
            enqueue(r, r)

        def round_body(k, carry):
            i0 = k * RING
            for r in range(RING):
                extract(i0 + r, r)
                enqueue(i0 + RING + r, r)
            return carry

        lax.fori_loop(0, ROUNDS - 1, round_body, 0)
        for r in range(RING):
            extract((ROUNDS - 1) * RING + r, r)

    gather_role(uid_hbm, uembT_hbm, ucols_v)
    gather_role(pid_hbm, iembT_hbm, pcols_v)
    gather_role(nid_hbm, iembT_hbm, ncols_v)

    def chunk(c, carry):
        col = c * LANES + lanes
        accp = jnp.zeros((LANES,), jnp.float32)
        accn = jnp.zeros((LANES,), jnp.float32)
        for d in range(DIM):
            dv = jnp.full((LANES,), d, jnp.int32)
            u = plsc.load_gather(ucols_v, [dv, col])
            p = plsc.load_gather(pcols_v, [dv, col])
            n = plsc.load_gather(ncols_v, [dv, col])
            accp = accp + u * p
            accn = accn + u * n
        outp_v[pl.ds(c * LANES, LANES)] = accp
        outn_v[pl.ds(c * LANES, LANES)] = accn
        return carry

    lax.fori_loop(0, CHUNKS, chunk, 0)

    pltpu.sync_copy(outp_v, outp_hbm.at[pl.ds(base, BPW)])
    pltpu.sync_copy(outn_v, outn_hbm.at[pl.ds(base, BPW)])


def kernel(user_ids, pos_item_ids, neg_item_ids, user_emb, item_emb):
    mesh = plsc.VectorSubcoreMesh(
        core_axis_name="c", subcore_axis_name="s",
        num_cores=NUM_CORES, num_subcores=NUM_SUBCORES)
    out_type = (jax.ShapeDtypeStruct((BATCH,), jnp.float32),
                jax.ShapeDtypeStruct((BATCH,), jnp.float32))
    scratch = [
        pltpu.SMEM((BPW,), jnp.int32),
        pltpu.VMEM((BPW,), jnp.int32),
        pltpu.VMEM((RING, DIM, 128), jnp.float32),
        pltpu.VMEM((DIM, BPW), jnp.float32),
        pltpu.VMEM((DIM, BPW), jnp.float32),
        pltpu.VMEM((DIM, BPW), jnp.float32),
        pltpu.VMEM((BPW,), jnp.float32),
        pltpu.VMEM((BPW,), jnp.float32),
        pltpu.SemaphoreType.DMA((RING,)),
    ]
    f = pl.kernel(_bpr_body, out_type=out_type, mesh=mesh,
                  scratch_types=scratch,
                  compiler_params=pltpu.CompilerParams(
                      needs_layout_passes=False,
                      use_tc_tiling_on_sc=True))
    return f(user_ids.astype(jnp.int32), pos_item_ids.astype(jnp.int32),
             neg_item_ids.astype(jnp.int32), user_emb.T, item_emb.T)


# speedup vs baseline: 1.0483x; 1.0483x over previous
import jax
import jax.numpy as jnp
from jax import lax
from jax.experimental import pallas as pl
from jax.experimental.pallas import tpu as pltpu
from jax.experimental.pallas import tpu_sc as plsc

NUM_CORES = 2
NUM_SUBCORES = 16
LANES = 16
NW = NUM_CORES * NUM_SUBCORES

BATCH = 16384
DIM = 32
BPW = BATCH // NW
RING = 8
ROUNDS = BPW // RING
CHUNKS = BPW // LANES


def _bpr_body(uid_hbm, pid_hbm, nid_hbm, uembT_hbm, iembT_hbm,
              outp_hbm, outn_hbm,
              ids_sm, ids_v, ring_v, ucols_v, pcols_v, ncols_v,
              outp_v, outn_v, sems):
    wid = lax.axis_index("s") * NUM_CORES + lax.axis_index("c")
    base = pl.multiple_of(wid * BPW, 128)
    lanes = lax.iota(jnp.int32, LANES)

    def gather_role(id_hbm, tab_hbm, cols_v):
        pltpu.sync_copy(id_hbm.at[pl.ds(base, BPW)], ids_v)

        def load_ids(k, carry):
            vec = ids_v[pl.ds(k * LANES, LANES)]
            for l in range(LANES):
                b = lax.reduce_max(jnp.where(lanes == l, vec, 0), (0,))
                ids_sm[k * LANES + l] = b
            return carry

        lax.fori_loop(0, BPW // LANES, load_ids, 0)

        def enqueue(i, r):
            blk = pl.multiple_of(ids_sm[i] & ~(128 - 1), 128)
            pltpu.async_copy(tab_hbm.at[:, pl.ds(blk, 128)],
                             ring_v.at[r], sems.at[r])

        def extract(i, r):
            pltpu.make_async_copy(tab_hbm.at[:, pl.ds(0, 128)],
                                  ring_v.at[r], sems.at[r]).wait()
            j = jnp.full((LANES,), ids_sm[i] & (128 - 1), jnp.int32)
            col = jnp.full((LANES,), i, jnp.int32)
            v0 = plsc.load_gather(ring_v.at[r], [lanes, j])
            v1 = plsc.load_gather(ring_v.at[r], [lanes + LANES, j])
            plsc.store_scatter(cols_v, [lanes, col], v0)
            plsc.store_scatter(cols_v, [lanes + LANES, col], v1)

        for r in range(RING):
            enqueue(r, r)

        def round_body(k, carry):
            i0 = k * RING
            for r in range(RING):
                extract(i0 + r, r)
                enqueue(i0 + RING + r, r)
            return carry

        lax.fori_loop(0, ROUNDS - 1, round_body, 0)
        for r in range(RING):
            extract((ROUNDS - 1) * RING + r, r)

    gather_role(uid_hbm, uembT_hbm, ucols_v)
    gather_role(pid_hbm, iembT_hbm, pcols_v)
    gather_role(nid_hbm, iembT_hbm, ncols_v)

    def chunk(c, carry):
        col = c * LANES + lanes
        accp = jnp.zeros((LANES,), jnp.float32)
        accn = jnp.zeros((LANES,), jnp.float32)
        for d in range(DIM):
            dv = jnp.full((LANES,), d, jnp.int32)
            u = plsc.load_gather(ucols_v, [dv, col])
            p = plsc.load_gather(pcols_v, [dv, col])
            n = plsc.load_gather(ncols_v, [dv, col])
            accp = accp + u * p
            accn = accn + u * n
        outp_v[pl.ds(c * LANES, LANES)] = accp
        outn_v[pl.ds(c * LANES, LANES)] = accn
        return carry

    lax.fori_loop(0, CHUNKS, chunk, 0)

    pltpu.sync_copy(outp_v, outp_hbm.at[pl.ds(base, BPW)])
    pltpu.sync_copy(outn_v, outn_hbm.at[pl.ds(base, BPW)])


def kernel(user_ids, pos_item_ids, neg_item_ids, user_emb, item_emb):
    mesh = plsc.VectorSubcoreMesh(
        core_axis_name="c", subcore_axis_name="s",
        num_cores=NUM_CORES, num_subcores=NUM_SUBCORES)
    out_type = (jax.ShapeDtypeStruct((BATCH,), jnp.float32),
                jax.ShapeDtypeStruct((BATCH,), jnp.float32))
    scratch = [
        pltpu.SMEM((BPW,), jnp.int32),
        pltpu.VMEM((BPW,), jnp.int32),
        pltpu.VMEM((RING, DIM, 128), jnp.float32),
        pltpu.VMEM((DIM, BPW), jnp.float32),
        pltpu.VMEM((DIM, BPW), jnp.float32),
        pltpu.VMEM((DIM, BPW), jnp.float32),
        pltpu.VMEM((BPW,), jnp.float32),
        pltpu.VMEM((BPW,), jnp.float32),
        pltpu.SemaphoreType.DMA((RING,)),
    ]
    f = pl.kernel(_bpr_body, out_type=out_type, mesh=mesh,
                  scratch_types=scratch,
                  compiler_params=pltpu.CompilerParams(
                      needs_layout_passes=False,
                      use_tc_tiling_on_sc=True))
    return f(user_ids.astype(jnp.int32), pos_item_ids.astype(jnp.int32),
             neg_item_ids.astype(jnp.int32), user_emb.T, item_emb.T)
